# baseline (speedup 1.0000x reference)
import functools

import jax
import jax.numpy as jnp
from jax import lax
from jax.experimental import pallas as pl
from jax.experimental.pallas import tpu as pltpu
from jax.experimental.pallas import tpu_sc as plsc

N = 10000
E = 320000
D = 128
HID = 128
NCLS = 10
HOPS = 2

NSC = 2
NT = 16
NW = NSC * NT
EPW = E // NW
K = 128
EPWP = 10240
CHP = EPWP // K
SLAB = CHP // 2
NPAD = 10240
RPT = NPAD // NT

_MESH = plsc.VectorSubcoreMesh(
    core_axis_name="c", subcore_axis_name="s", num_cores=NSC, num_subcores=NT
)


WD = 8


@functools.partial(
    pl.kernel,
    out_type=jax.ShapeDtypeStruct((NSC, NPAD, WD), jnp.float32),
    mesh=_MESH,
    scratch_types=[
        pltpu.VMEM((CHP, K), jnp.int32),
        pltpu.VMEM((K, WD), jnp.float32),
        pltpu.VMEM_SHARED((NPAD, WD), jnp.float32),
        pltpu.SemaphoreType.DMA,
    ],
    compiler_params=pltpu.CompilerParams(use_tc_tiling_on_sc=False),
)
def _sc_deg(dst_hbm, ones_hbm, zero_hbm, out_hbm, dst_v, ones_v, accd, sem):
    c = lax.axis_index("c")
    s = lax.axis_index("s")
    w = c * NT + s
    r0 = s * RPT
    pltpu.sync_copy(zero_hbm.at[pl.ds(r0, RPT)], accd.at[pl.ds(r0, RPT)])
    pltpu.sync_copy(dst_hbm.at[w], dst_v)
    pltpu.sync_copy(ones_hbm, ones_v)
    plsc.subcore_barrier()

    def chunk(i, carry):
        pltpu.sync_copy(ones_v, accd.at[dst_v.at[i]], add=True)
        return carry

    lax.fori_loop(0, CHP, chunk, 0)
    plsc.subcore_barrier()
    pltpu.sync_copy(accd.at[pl.ds(r0, RPT)], out_hbm.at[c, pl.ds(r0, RPT)])


@functools.partial(
    pl.kernel,
    out_type=jax.ShapeDtypeStruct((NSC, NPAD, D), jnp.float32),
    mesh=_MESH,
    scratch_types=[
        pltpu.VMEM((SLAB, K), jnp.int32),
        pltpu.VMEM((SLAB, K), jnp.int32),
        pltpu.VMEM((K, D), jnp.float32),
        pltpu.VMEM((K, D), jnp.float32),
        pltpu.VMEM_SHARED((NPAD, D), jnp.float32),
        pltpu.SemaphoreType.DMA,
        pltpu.SemaphoreType.DMA,
    ],
    compiler_params=pltpu.CompilerParams(use_tc_tiling_on_sc=False),
)
def _sc_hop(g_hbm, src_hbm, dst_hbm, zero_hbm, out_hbm,
            src_v, dst_v, rows_a, rows_b, acc, sem_a, sem_b):
    c = lax.axis_index("c")
    s = lax.axis_index("s")
    w = c * NT + s
    r0 = s * RPT
    pltpu.sync_copy(zero_hbm.at[pl.ds(r0, RPT)], acc.at[pl.ds(r0, RPT)])
    plsc.subcore_barrier()

    for half in range(CHP // SLAB):
        pltpu.sync_copy(src_hbm.at[w, pl.ds(half * SLAB, SLAB)], src_v)
        pltpu.sync_copy(dst_hbm.at[w, pl.ds(half * SLAB, SLAB)], dst_v)

        pltpu.async_copy(g_hbm.at[src_v.at[0]], rows_a, sem_a)

        def pair(j, carry):
            i0 = 2 * j
            i1 = i0 + 1
            pltpu.make_async_copy(g_hbm.at[src_v.at[i0]], rows_a, sem_a).wait()
            pltpu.async_copy(g_hbm.at[src_v.at[i1]], rows_b, sem_b)
            pltpu.sync_copy(rows_a, acc.at[dst_v.at[i0]], add=True)
            pltpu.make_async_copy(g_hbm.at[src_v.at[i1]], rows_b, sem_b).wait()
            pltpu.async_copy(g_hbm.at[src_v.at[i0 + 2]], rows_a, sem_a)
            pltpu.sync_copy(rows_b, acc.at[dst_v.at[i1]], add=True)
            return carry

        lax.fori_loop(0, (SLAB - 2) // 2, pair, 0)
        pltpu.make_async_copy(g_hbm.at[src_v.at[SLAB - 2]], rows_a, sem_a).wait()
        pltpu.async_copy(g_hbm.at[src_v.at[SLAB - 1]], rows_b, sem_b)
        pltpu.sync_copy(rows_a, acc.at[dst_v.at[SLAB - 2]], add=True)
        pltpu.make_async_copy(g_hbm.at[src_v.at[SLAB - 1]], rows_b, sem_b).wait()
        pltpu.sync_copy(rows_b, acc.at[dst_v.at[SLAB - 1]], add=True)

    plsc.subcore_barrier()
    pltpu.sync_copy(acc.at[pl.ds(r0, RPT)], out_hbm.at[c, pl.ds(r0, RPT)])


_R = 1000


def _tc_norm(degp):
    def body(degp_ref, norm_ref):
        deg = degp_ref[0] + degp_ref[1]
        nrm = jnp.where(deg > 0.0, lax.rsqrt(jnp.maximum(deg, 1.0)), 0.0)
        norm_ref[...] = nrm[0:N, 0:1]

    return pl.pallas_call(
        body,
        out_shape=jax.ShapeDtypeStruct((N, 1), jnp.float32),
    )(degp)


def _tc_scale(x, norm_c):
    def body(x_ref, n_ref, g_ref):
        g_ref[...] = x_ref[...] * n_ref[...]

    return pl.pallas_call(
        body,
        grid=(N // _R,),
        in_specs=[
            pl.BlockSpec((_R, D), lambda r: (r, 0)),
            pl.BlockSpec((_R, 1), lambda r: (r, 0)),
        ],
        out_specs=pl.BlockSpec((_R, D), lambda r: (r, 0)),
        out_shape=jax.ShapeDtypeStruct((N, D), jnp.float32),
    )(x, norm_c)


def _tc_combine(p, norm_c):
    def body(p_ref, n_ref, c_ref, g_ref):
        nb = n_ref[...]
        cb = (p_ref[0] + p_ref[1]) * nb
        c_ref[...] = cb
        g_ref[...] = cb * nb

    return pl.pallas_call(
        body,
        grid=(N // _R,),
        in_specs=[
            pl.BlockSpec((NSC, _R, D), lambda r: (0, r, 0)),
            pl.BlockSpec((_R, 1), lambda r: (r, 0)),
        ],
        out_specs=[pl.BlockSpec((_R, D), lambda r: (r, 0))] * 2,
        out_shape=[jax.ShapeDtypeStruct((N, D), jnp.float32)] * 2,
    )(p, norm_c)


def _tc_layer1(x, c1, p, norm_c, W, b):
    def body(x_ref, c1_ref, p_ref, n_ref, w_ref, b_ref, h_ref, g_ref):
        nb = n_ref[...]
        c2 = (p_ref[0] + p_ref[1]) * nb
        z = jnp.dot(x_ref[...], w_ref[0:D], preferred_element_type=jnp.float32)
        z += jnp.dot(c1_ref[...], w_ref[D:2 * D], preferred_element_type=jnp.float32)
        z += jnp.dot(c2, w_ref[2 * D:3 * D], preferred_element_type=jnp.float32)
        h = jnp.maximum(z + b_ref[...], 0.0)
        h_ref[...] = h
        g_ref[...] = h * nb

    return pl.pallas_call(
        body,
        grid=(N // _R,),
        in_specs=[
            pl.BlockSpec((_R, D), lambda r: (r, 0)),
            pl.BlockSpec((_R, D), lambda r: (r, 0)),
            pl.BlockSpec((NSC, _R, D), lambda r: (0, r, 0)),
            pl.BlockSpec((_R, 1), lambda r: (r, 0)),
            pl.BlockSpec(((HOPS + 1) * D, HID), lambda r: (0, 0)),
            pl.BlockSpec((1, HID), lambda r: (0, 0)),
        ],
        out_specs=[pl.BlockSpec((_R, D), lambda r: (r, 0))] * 2,
        out_shape=[jax.ShapeDtypeStruct((N, HID), jnp.float32)] * 2,
    )(x, c1, p, norm_c, W, b)


def _tc_layer2(h1, c1, p, norm_c, W, b, Wc, bc):
    G = N // _R

    def body(h1_ref, c1_ref, p_ref, n_ref, w_ref, b_ref, wc_ref, bc_ref,
             out_ref, acc_ref):
        r = pl.program_id(0)
        nb = n_ref[...]
        c2 = (p_ref[0] + p_ref[1]) * nb
        z = jnp.dot(h1_ref[...], w_ref[0:D], preferred_element_type=jnp.float32)
        z += jnp.dot(c1_ref[...], w_ref[D:2 * D], preferred_element_type=jnp.float32)
        z += jnp.dot(c2, w_ref[2 * D:3 * D], preferred_element_type=jnp.float32)
        h = jnp.maximum(z + b_ref[...], 0.0)
        ssum = jnp.sum(h, axis=0, keepdims=True)

        @pl.when(r == 0)
        def _():
            acc_ref[...] = ssum

        @pl.when(r != 0)
        def _():
            acc_ref[...] = acc_ref[...] + ssum

        @pl.when(r == G - 1)
        def _():
            hg = acc_ref[...] * (1.0 / N)
            out_ref[...] = (
                jnp.dot(hg, wc_ref[...], preferred_element_type=jnp.float32)
                + bc_ref[...]
            )

    return pl.pallas_call(
        body,
        grid=(G,),
        in_specs=[
            pl.BlockSpec((_R, HID), lambda r: (r, 0)),
            pl.BlockSpec((_R, HID), lambda r: (r, 0)),
            pl.BlockSpec((NSC, _R, HID), lambda r: (0, r, 0)),
            pl.BlockSpec((_R, 1), lambda r: (r, 0)),
            pl.BlockSpec(((HOPS + 1) * HID, HID), lambda r: (0, 0)),
            pl.BlockSpec((1, HID), lambda r: (0, 0)),
            pl.BlockSpec((HID, NCLS), lambda r: (0, 0)),
            pl.BlockSpec((1, NCLS), lambda r: (0, 0)),
        ],
        out_specs=pl.BlockSpec((1, NCLS), lambda r: (0, 0)),
        out_shape=jax.ShapeDtypeStruct((1, NCLS), jnp.float32),
        scratch_shapes=[pltpu.VMEM((1, HID), jnp.float32)],
    )(h1, c1, p, norm_c, W, b, Wc, bc)


def kernel(x, edge_index, W1, b1, W2, b2, Wc, bc):
    pad = NW * EPWP - E
    srcp = jnp.concatenate(
        [edge_index[0], jnp.zeros((pad,), jnp.int32)]).reshape(NW, CHP, K)
    dstp = jnp.concatenate(
        [edge_index[1], jnp.full((pad,), N, jnp.int32)]).reshape(NW, CHP, K)
    zeros2d = jnp.zeros((NPAD, D), jnp.float32)

    degp = _sc_deg(dstp, jnp.ones((K, WD), jnp.float32),
                   jnp.zeros((NPAD, WD), jnp.float32))
    norm_c = _tc_norm(degp)

    g = _tc_scale(x, norm_c)
    pA = _sc_hop(g, srcp, dstp, zeros2d)
    c1, g = _tc_combine(pA, norm_c)
    pB = _sc_hop(g, srcp, dstp, zeros2d)
    h1, g = _tc_layer1(x, c1, pB, norm_c, W1, b1.reshape(1, HID))
    pC = _sc_hop(g, srcp, dstp, zeros2d)
    c1b, g = _tc_combine(pC, norm_c)
    pD = _sc_hop(g, srcp, dstp, zeros2d)
    return _tc_layer2(h1, c1b, pD, norm_c, W2, b2.reshape(1, HID),
                      Wc, bc.reshape(1, NCLS))


# speedup vs baseline: 1.0024x; 1.0024x over previous
import functools

import jax
import jax.numpy as jnp
from jax import lax
from jax.experimental import pallas as pl
from jax.experimental.pallas import tpu as pltpu
from jax.experimental.pallas import tpu_sc as plsc

N = 10000
E = 320000
D = 128
HID = 128
NCLS = 10
HOPS = 2

NSC = 2
NT = 16
NW = NSC * NT
EPW = E // NW
K = 128
EPWP = 10240
CHP = EPWP // K
SLAB = CHP // 2
NPAD = 10240
RPT = NPAD // NT

_MESH = plsc.VectorSubcoreMesh(
    core_axis_name="c", subcore_axis_name="s", num_cores=NSC, num_subcores=NT
)


WD = 8


@functools.partial(
    pl.kernel,
    out_type=jax.ShapeDtypeStruct((NSC, NPAD, WD), jnp.float32),
    mesh=_MESH,
    scratch_types=[
        pltpu.VMEM((CHP, K), jnp.int32),
        pltpu.VMEM((K, WD), jnp.float32),
        pltpu.VMEM_SHARED((NPAD, WD), jnp.float32),
        pltpu.SemaphoreType.DMA,
    ],
    compiler_params=pltpu.CompilerParams(use_tc_tiling_on_sc=False),
)
def _sc_deg(dst_hbm, ones_hbm, zero_hbm, out_hbm, dst_v, ones_v, accd, sem):
    c = lax.axis_index("c")
    s = lax.axis_index("s")
    w = c * NT + s
    r0 = s * RPT
    pltpu.sync_copy(zero_hbm.at[pl.ds(r0, RPT)], accd.at[pl.ds(r0, RPT)])
    pltpu.sync_copy(dst_hbm.at[w], dst_v)
    pltpu.sync_copy(ones_hbm, ones_v)
    plsc.subcore_barrier()

    def chunk(i, carry):
        pltpu.sync_copy(ones_v, accd.at[dst_v.at[i]], add=True)
        return carry

    lax.fori_loop(0, CHP, chunk, 0)
    plsc.subcore_barrier()
    pltpu.sync_copy(accd.at[pl.ds(r0, RPT)], out_hbm.at[c, pl.ds(r0, RPT)])


@functools.partial(
    pl.kernel,
    out_type=jax.ShapeDtypeStruct((NSC, NPAD, D), jnp.float32),
    mesh=_MESH,
    scratch_types=[
        pltpu.VMEM((SLAB, K), jnp.int32),
        pltpu.VMEM((SLAB, K), jnp.int32),
        pltpu.VMEM((K, D), jnp.float32),
        pltpu.VMEM((K, D), jnp.float32),
        pltpu.VMEM_SHARED((NPAD, D), jnp.float32),
        pltpu.SemaphoreType.DMA,
        pltpu.SemaphoreType.DMA,
    ],
    compiler_params=pltpu.CompilerParams(use_tc_tiling_on_sc=False),
)
def _sc_hop(g_hbm, src_hbm, dst_hbm, zero_hbm, out_hbm,
            src_v, dst_v, rows_a, rows_b, acc, sem_a, sem_b):
    c = lax.axis_index("c")
    s = lax.axis_index("s")
    w = c * NT + s
    r0 = s * RPT
    pltpu.sync_copy(zero_hbm.at[pl.ds(r0, RPT)], acc.at[pl.ds(r0, RPT)])
    plsc.subcore_barrier()

    for half in range(CHP // SLAB):
        pltpu.sync_copy(src_hbm.at[w, pl.ds(half * SLAB, SLAB)], src_v)
        pltpu.sync_copy(dst_hbm.at[w, pl.ds(half * SLAB, SLAB)], dst_v)

        pltpu.async_copy(g_hbm.at[src_v.at[0]], rows_a, sem_a)

        def pair(j, carry):
            i0 = 2 * j
            i1 = i0 + 1
            pltpu.make_async_copy(g_hbm.at[src_v.at[i0]], rows_a, sem_a).wait()
            pltpu.async_copy(g_hbm.at[src_v.at[i1]], rows_b, sem_b)
            pltpu.sync_copy(rows_a, acc.at[dst_v.at[i0]], add=True)
            pltpu.make_async_copy(g_hbm.at[src_v.at[i1]], rows_b, sem_b).wait()
            pltpu.async_copy(g_hbm.at[src_v.at[i0 + 2]], rows_a, sem_a)
            pltpu.sync_copy(rows_b, acc.at[dst_v.at[i1]], add=True)
            return carry

        lax.fori_loop(0, (SLAB - 2) // 2, pair, 0)
        pltpu.make_async_copy(g_hbm.at[src_v.at[SLAB - 2]], rows_a, sem_a).wait()
        pltpu.async_copy(g_hbm.at[src_v.at[SLAB - 1]], rows_b, sem_b)
        pltpu.sync_copy(rows_a, acc.at[dst_v.at[SLAB - 2]], add=True)
        pltpu.make_async_copy(g_hbm.at[src_v.at[SLAB - 1]], rows_b, sem_b).wait()
        pltpu.sync_copy(rows_b, acc.at[dst_v.at[SLAB - 1]], add=True)

    plsc.subcore_barrier()
    pltpu.sync_copy(acc.at[pl.ds(r0, RPT)], out_hbm.at[c, pl.ds(r0, RPT)])


_R = 1000


def _tc_norm(degp):
    def body(degp_ref, norm_ref):
        deg = degp_ref[0] + degp_ref[1]
        nrm = jnp.where(deg > 0.0, lax.rsqrt(jnp.maximum(deg, 1.0)), 0.0)
        norm_ref[...] = nrm[0:N, 0:1]

    return pl.pallas_call(
        body,
        out_shape=jax.ShapeDtypeStruct((N, 1), jnp.float32),
    )(degp)


def _tc_scale(x, norm_c):
    def body(x_ref, n_ref, g_ref):
        g_ref[...] = x_ref[...] * n_ref[...]

    return pl.pallas_call(
        body,
        grid=(N // _R,),
        in_specs=[
            pl.BlockSpec((_R, D), lambda r: (r, 0)),
            pl.BlockSpec((_R, 1), lambda r: (r, 0)),
        ],
        out_specs=pl.BlockSpec((_R, D), lambda r: (r, 0)),
        out_shape=jax.ShapeDtypeStruct((N, D), jnp.float32),
    )(x, norm_c)


def _tc_combine(p, norm_c):
    def body(p_ref, n_ref, c_ref, g_ref):
        nb = n_ref[...]
        cb = (p_ref[0] + p_ref[1]) * nb
        c_ref[...] = cb
        g_ref[...] = cb * nb

    return pl.pallas_call(
        body,
        grid=(N // _R,),
        in_specs=[
            pl.BlockSpec((NSC, _R, D), lambda r: (0, r, 0)),
            pl.BlockSpec((_R, 1), lambda r: (r, 0)),
        ],
        out_specs=[pl.BlockSpec((_R, D), lambda r: (r, 0))] * 2,
        out_shape=[jax.ShapeDtypeStruct((N, D), jnp.float32)] * 2,
    )(p, norm_c)


def _tc_layer1(x, c1, p, norm_c, W, b):
    def body(x_ref, c1_ref, p_ref, n_ref, w_ref, b_ref, h_ref, g_ref):
        nb = n_ref[...]
        c2 = (p_ref[0] + p_ref[1]) * nb
        z = jnp.dot(x_ref[...], w_ref[0:D], preferred_element_type=jnp.float32)
        z += jnp.dot(c1_ref[...], w_ref[D:2 * D], preferred_element_type=jnp.float32)
        z += jnp.dot(c2, w_ref[2 * D:3 * D], preferred_element_type=jnp.float32)
        h = jnp.maximum(z + b_ref[...], 0.0)
        h_ref[...] = h
        g_ref[...] = h * nb

    return pl.pallas_call(
        body,
        grid=(N // _R,),
        in_specs=[
            pl.BlockSpec((_R, D), lambda r: (r, 0)),
            pl.BlockSpec((_R, D), lambda r: (r, 0)),
            pl.BlockSpec((NSC, _R, D), lambda r: (0, r, 0)),
            pl.BlockSpec((_R, 1), lambda r: (r, 0)),
            pl.BlockSpec(((HOPS + 1) * D, HID), lambda r: (0, 0)),
            pl.BlockSpec((1, HID), lambda r: (0, 0)),
        ],
        out_specs=[pl.BlockSpec((_R, D), lambda r: (r, 0))] * 2,
        out_shape=[jax.ShapeDtypeStruct((N, HID), jnp.float32)] * 2,
    )(x, c1, p, norm_c, W, b)


def _tc_layer2(h1, c1, p, norm_c, W, b, Wc, bc):
    G = N // _R

    def body(h1_ref, c1_ref, p_ref, n_ref, w_ref, b_ref, wc_ref, bc_ref,
             out_ref, acc_ref):
        r = pl.program_id(0)
        nb = n_ref[...]
        c2 = (p_ref[0] + p_ref[1]) * nb
        z = jnp.dot(h1_ref[...], w_ref[0:D], preferred_element_type=jnp.float32)
        z += jnp.dot(c1_ref[...], w_ref[D:2 * D], preferred_element_type=jnp.float32)
        z += jnp.dot(c2, w_ref[2 * D:3 * D], preferred_element_type=jnp.float32)
        h = jnp.maximum(z + b_ref[...], 0.0)
        ssum = jnp.sum(h, axis=0, keepdims=True)

        @pl.when(r == 0)
        def _():
            acc_ref[...] = ssum

        @pl.when(r != 0)
        def _():
            acc_ref[...] = acc_ref[...] + ssum

        @pl.when(r == G - 1)
        def _():
            hg = acc_ref[...] * (1.0 / N)
            out_ref[...] = (
                jnp.dot(hg, wc_ref[...], preferred_element_type=jnp.float32)
                + bc_ref[...]
            )

    return pl.pallas_call(
        body,
        grid=(G,),
        in_specs=[
            pl.BlockSpec((_R, HID), lambda r: (r, 0)),
            pl.BlockSpec((_R, HID), lambda r: (r, 0)),
            pl.BlockSpec((NSC, _R, HID), lambda r: (0, r, 0)),
            pl.BlockSpec((_R, 1), lambda r: (r, 0)),
            pl.BlockSpec(((HOPS + 1) * HID, HID), lambda r: (0, 0)),
            pl.BlockSpec((1, HID), lambda r: (0, 0)),
            pl.BlockSpec((HID, NCLS), lambda r: (0, 0)),
            pl.BlockSpec((1, NCLS), lambda r: (0, 0)),
        ],
        out_specs=pl.BlockSpec((1, NCLS), lambda r: (0, 0)),
        out_shape=jax.ShapeDtypeStruct((1, NCLS), jnp.float32),
        scratch_shapes=[pltpu.VMEM((1, HID), jnp.float32)],
    )(h1, c1, p, norm_c, W, b, Wc, bc)


def kernel(x, edge_index, W1, b1, W2, b2, Wc, bc):
    pad = NW * EPWP - E
    srcp = jnp.concatenate(
        [edge_index[0], jnp.zeros((pad,), jnp.int32)]).reshape(NW, CHP, K)
    pad_dst = N + jnp.arange(pad, dtype=jnp.int32) % (NPAD - N)
    dstp = jnp.concatenate(
        [edge_index[1], pad_dst]).reshape(NW, CHP, K)
    zeros2d = jnp.zeros((NPAD, D), jnp.float32)

    degp = _sc_deg(dstp, jnp.ones((K, WD), jnp.float32),
                   jnp.zeros((NPAD, WD), jnp.float32))
    norm_c = _tc_norm(degp)

    g = _tc_scale(x, norm_c)
    pA = _sc_hop(g, srcp, dstp, zeros2d)
    c1, g = _tc_combine(pA, norm_c)
    pB = _sc_hop(g, srcp, dstp, zeros2d)
    h1, g = _tc_layer1(x, c1, pB, norm_c, W1, b1.reshape(1, HID))
    pC = _sc_hop(g, srcp, dstp, zeros2d)
    c1b, g = _tc_combine(pC, norm_c)
    pD = _sc_hop(g, srcp, dstp, zeros2d)
    return _tc_layer2(h1, c1b, pD, norm_c, W2, b2.reshape(1, HID),
                      Wc, bc.reshape(1, NCLS))


# speedup vs baseline: 3.0142x; 3.0069x over previous
import functools

import jax
import jax.numpy as jnp
from jax import lax
from jax.experimental import pallas as pl
from jax.experimental.pallas import tpu as pltpu
from jax.experimental.pallas import tpu_sc as plsc

N = 10000
E = 320000
D = 128
HID = 128
NCLS = 10
HOPS = 2

NSC = 2
NT = 16
NW = NSC * NT
EPW = E // NW
K = 40
CH = EPW // K
NBUF = 5
KD = 128
EPWP = 10240
CHD = EPWP // KD
NPAD = 10240
RPT = NPAD // NT

_MESH = plsc.VectorSubcoreMesh(
    core_axis_name="c", subcore_axis_name="s", num_cores=NSC, num_subcores=NT
)


WD = 8


@functools.partial(
    pl.kernel,
    out_type=jax.ShapeDtypeStruct((NSC, NPAD, WD), jnp.float32),
    mesh=_MESH,
    scratch_types=[
        pltpu.VMEM((CHD, KD), jnp.int32),
        pltpu.VMEM((KD, WD), jnp.float32),
        pltpu.VMEM_SHARED((NPAD, WD), jnp.float32),
        pltpu.SemaphoreType.DMA,
    ],
    compiler_params=pltpu.CompilerParams(use_tc_tiling_on_sc=False),
)
def _sc_deg(dst_hbm, ones_hbm, zero_hbm, out_hbm, dst_v, ones_v, accd, sem):
    c = lax.axis_index("c")
    s = lax.axis_index("s")
    w = c * NT + s
    r0 = s * RPT
    pltpu.sync_copy(zero_hbm.at[pl.ds(r0, RPT)], accd.at[pl.ds(r0, RPT)])
    pltpu.sync_copy(dst_hbm.at[w], dst_v)
    pltpu.sync_copy(ones_hbm, ones_v)
    plsc.subcore_barrier()

    def chunk(i, carry):
        pltpu.sync_copy(ones_v, accd.at[dst_v.at[i]], add=True)
        return carry

    lax.fori_loop(0, CHD, chunk, 0)
    plsc.subcore_barrier()
    pltpu.sync_copy(accd.at[pl.ds(r0, RPT)], out_hbm.at[c, pl.ds(r0, RPT)])


@functools.partial(
    pl.kernel,
    out_type=jax.ShapeDtypeStruct((NSC, NPAD, D), jnp.float32),
    mesh=_MESH,
    scratch_types=[
        pltpu.VMEM((CH, K), jnp.int32),
        pltpu.VMEM((CH, K), jnp.int32),
    ]
    + [pltpu.VMEM((K, D), jnp.float32)] * NBUF
    + [pltpu.SemaphoreType.DMA] * (2 * NBUF)
    + [pltpu.VMEM_SHARED((NPAD, D), jnp.float32)],
    compiler_params=pltpu.CompilerParams(use_tc_tiling_on_sc=False),
)
def _sc_hop(g_hbm, src_hbm, dst_hbm, zero_hbm, out_hbm, src_v, dst_v, *ring):
    rows = ring[:NBUF]
    gs = ring[NBUF:2 * NBUF]
    ss = ring[2 * NBUF:3 * NBUF]
    acc = ring[3 * NBUF]
    c = lax.axis_index("c")
    s = lax.axis_index("s")
    w = c * NT + s
    r0 = s * RPT
    pltpu.sync_copy(zero_hbm.at[pl.ds(r0, RPT)], acc.at[pl.ds(r0, RPT)])
    pltpu.sync_copy(src_hbm.at[w], src_v)
    pltpu.sync_copy(dst_hbm.at[w], dst_v)
    plsc.subcore_barrier()

    for b in range(NBUF):
        pltpu.async_copy(g_hbm.at[src_v.at[b]], rows[b], gs[b])

    def group(t, carry):
        base = t * NBUF
        for b in range(NBUF):
            i = base + b
            pltpu.make_async_copy(g_hbm.at[src_v.at[i]], rows[b], gs[b]).wait()
            pltpu.async_copy(rows[b], acc.at[dst_v.at[i]], ss[b], add=True)
        for b in range(NBUF):
            i = base + b
            pltpu.make_async_copy(rows[b], acc.at[dst_v.at[i]], ss[b]).wait()
            pltpu.async_copy(g_hbm.at[src_v.at[i + NBUF]], rows[b], gs[b])
        return carry

    lax.fori_loop(0, CH // NBUF - 1, group, 0)
    base = CH - NBUF
    for b in range(NBUF):
        i = base + b
        pltpu.make_async_copy(g_hbm.at[src_v.at[i]], rows[b], gs[b]).wait()
        pltpu.async_copy(rows[b], acc.at[dst_v.at[i]], ss[b], add=True)
    for b in range(NBUF):
        i = base + b
        pltpu.make_async_copy(rows[b], acc.at[dst_v.at[i]], ss[b]).wait()

    plsc.subcore_barrier()
    pltpu.sync_copy(acc.at[pl.ds(r0, RPT)], out_hbm.at[c, pl.ds(r0, RPT)])


_R = 1000


def _tc_norm(degp):
    def body(degp_ref, norm_ref):
        deg = degp_ref[0] + degp_ref[1]
        nrm = jnp.where(deg > 0.0, lax.rsqrt(jnp.maximum(deg, 1.0)), 0.0)
        norm_ref[...] = nrm[0:N, 0:1]

    return pl.pallas_call(
        body,
        out_shape=jax.ShapeDtypeStruct((N, 1), jnp.float32),
    )(degp)


def _tc_scale(x, norm_c):
    def body(x_ref, n_ref, g_ref):
        g_ref[...] = x_ref[...] * n_ref[...]

    return pl.pallas_call(
        body,
        grid=(N // _R,),
        in_specs=[
            pl.BlockSpec((_R, D), lambda r: (r, 0)),
            pl.BlockSpec((_R, 1), lambda r: (r, 0)),
        ],
        out_specs=pl.BlockSpec((_R, D), lambda r: (r, 0)),
        out_shape=jax.ShapeDtypeStruct((N, D), jnp.float32),
    )(x, norm_c)


def _tc_combine(p, norm_c):
    def body(p_ref, n_ref, c_ref, g_ref):
        nb = n_ref[...]
        cb = (p_ref[0] + p_ref[1]) * nb
        c_ref[...] = cb
        g_ref[...] = cb * nb

    return pl.pallas_call(
        body,
        grid=(N // _R,),
        in_specs=[
            pl.BlockSpec((NSC, _R, D), lambda r: (0, r, 0)),
            pl.BlockSpec((_R, 1), lambda r: (r, 0)),
        ],
        out_specs=[pl.BlockSpec((_R, D), lambda r: (r, 0))] * 2,
        out_shape=[jax.ShapeDtypeStruct((N, D), jnp.float32)] * 2,
    )(p, norm_c)


def _tc_layer1(x, c1, p, norm_c, W, b):
    def body(x_ref, c1_ref, p_ref, n_ref, w_ref, b_ref, h_ref, g_ref):
        nb = n_ref[...]
        c2 = (p_ref[0] + p_ref[1]) * nb
        z = jnp.dot(x_ref[...], w_ref[0:D], preferred_element_type=jnp.float32)
        z += jnp.dot(c1_ref[...], w_ref[D:2 * D], preferred_element_type=jnp.float32)
        z += jnp.dot(c2, w_ref[2 * D:3 * D], preferred_element_type=jnp.float32)
        h = jnp.maximum(z + b_ref[...], 0.0)
        h_ref[...] = h
        g_ref[...] = h * nb

    return pl.pallas_call(
        body,
        grid=(N // _R,),
        in_specs=[
            pl.BlockSpec((_R, D), lambda r: (r, 0)),
            pl.BlockSpec((_R, D), lambda r: (r, 0)),
            pl.BlockSpec((NSC, _R, D), lambda r: (0, r, 0)),
            pl.BlockSpec((_R, 1), lambda r: (r, 0)),
            pl.BlockSpec(((HOPS + 1) * D, HID), lambda r: (0, 0)),
            pl.BlockSpec((1, HID), lambda r: (0, 0)),
        ],
        out_specs=[pl.BlockSpec((_R, D), lambda r: (r, 0))] * 2,
        out_shape=[jax.ShapeDtypeStruct((N, HID), jnp.float32)] * 2,
    )(x, c1, p, norm_c, W, b)


def _tc_layer2(h1, c1, p, norm_c, W, b, Wc, bc):
    G = N // _R

    def body(h1_ref, c1_ref, p_ref, n_ref, w_ref, b_ref, wc_ref, bc_ref,
             out_ref, acc_ref):
        r = pl.program_id(0)
        nb = n_ref[...]
        c2 = (p_ref[0] + p_ref[1]) * nb
        z = jnp.dot(h1_ref[...], w_ref[0:D], preferred_element_type=jnp.float32)
        z += jnp.dot(c1_ref[...], w_ref[D:2 * D], preferred_element_type=jnp.float32)
        z += jnp.dot(c2, w_ref[2 * D:3 * D], preferred_element_type=jnp.float32)
        h = jnp.maximum(z + b_ref[...], 0.0)
        ssum = jnp.sum(h, axis=0, keepdims=True)

        @pl.when(r == 0)
        def _():
            acc_ref[...] = ssum

        @pl.when(r != 0)
        def _():
            acc_ref[...] = acc_ref[...] + ssum

        @pl.when(r == G - 1)
        def _():
            hg = acc_ref[...] * (1.0 / N)
            out_ref[...] = (
                jnp.dot(hg, wc_ref[...], preferred_element_type=jnp.float32)
                + bc_ref[...]
            )

    return pl.pallas_call(
        body,
        grid=(G,),
        in_specs=[
            pl.BlockSpec((_R, HID), lambda r: (r, 0)),
            pl.BlockSpec((_R, HID), lambda r: (r, 0)),
            pl.BlockSpec((NSC, _R, HID), lambda r: (0, r, 0)),
            pl.BlockSpec((_R, 1), lambda r: (r, 0)),
            pl.BlockSpec(((HOPS + 1) * HID, HID), lambda r: (0, 0)),
            pl.BlockSpec((1, HID), lambda r: (0, 0)),
            pl.BlockSpec((HID, NCLS), lambda r: (0, 0)),
            pl.BlockSpec((1, NCLS), lambda r: (0, 0)),
        ],
        out_specs=pl.BlockSpec((1, NCLS), lambda r: (0, 0)),
        out_shape=jax.ShapeDtypeStruct((1, NCLS), jnp.float32),
        scratch_shapes=[pltpu.VMEM((1, HID), jnp.float32)],
    )(h1, c1, p, norm_c, W, b, Wc, bc)


def kernel(x, edge_index, W1, b1, W2, b2, Wc, bc):
    src = edge_index[0].reshape(NW, CH, K)
    dst = edge_index[1].reshape(NW, CH, K)
    pad = NW * EPWP - E
    pad_dst = N + jnp.arange(pad, dtype=jnp.int32) % (NPAD - N)
    dstp = jnp.concatenate([edge_index[1], pad_dst]).reshape(NW, CHD, KD)
    zeros2d = jnp.zeros((NPAD, D), jnp.float32)

    degp = _sc_deg(dstp, jnp.ones((KD, WD), jnp.float32),
                   jnp.zeros((NPAD, WD), jnp.float32))
    norm_c = _tc_norm(degp)

    g = _tc_scale(x, norm_c)
    pA = _sc_hop(g, src, dst, zeros2d)
    c1, g = _tc_combine(pA, norm_c)
    pB = _sc_hop(g, src, dst, zeros2d)
    h1, g = _tc_layer1(x, c1, pB, norm_c, W1, b1.reshape(1, HID))
    pC = _sc_hop(g, src, dst, zeros2d)
    c1b, g = _tc_combine(pC, norm_c)
    pD = _sc_hop(g, src, dst, zeros2d)
    return _tc_layer2(h1, c1b, pD, norm_c, W2, b2.reshape(1, HID),
                      Wc, bc.reshape(1, NCLS))


# speedup vs baseline: 3.8650x; 1.2822x over previous
import functools

import jax
import jax.numpy as jnp
from jax import lax
from jax.experimental import pallas as pl
from jax.experimental.pallas import tpu as pltpu
from jax.experimental.pallas import tpu_sc as plsc

N = 10000
E = 320000
D = 128
HID = 128
NCLS = 10
HOPS = 2

NSC = 2
NT = 16
NW = NSC * NT
EPW = E // NW
K = 40
CH = EPW // K
NBUF = 10
DH = D // 2
KD = 128
EPWP = 10240
CHD = EPWP // KD
NPAD = 10240
RPT = NPAD // NT

_MESH = plsc.VectorSubcoreMesh(
    core_axis_name="c", subcore_axis_name="s", num_cores=NSC, num_subcores=NT
)


WD = 8


@functools.partial(
    pl.kernel,
    out_type=jax.ShapeDtypeStruct((NSC, NPAD, WD), jnp.float32),
    mesh=_MESH,
    scratch_types=[
        pltpu.VMEM((CHD, KD), jnp.int32),
        pltpu.VMEM((KD, WD), jnp.float32),
        pltpu.VMEM_SHARED((NPAD, WD), jnp.float32),
        pltpu.SemaphoreType.DMA,
    ],
    compiler_params=pltpu.CompilerParams(use_tc_tiling_on_sc=False),
)
def _sc_deg(dst_hbm, ones_hbm, zero_hbm, out_hbm, dst_v, ones_v, accd, sem):
    c = lax.axis_index("c")
    s = lax.axis_index("s")
    w = c * NT + s
    r0 = s * RPT
    pltpu.sync_copy(zero_hbm.at[pl.ds(r0, RPT)], accd.at[pl.ds(r0, RPT)])
    pltpu.sync_copy(dst_hbm.at[w], dst_v)
    pltpu.sync_copy(ones_hbm, ones_v)
    plsc.subcore_barrier()

    def chunk(i, carry):
        pltpu.sync_copy(ones_v, accd.at[dst_v.at[i]], add=True)
        return carry

    lax.fori_loop(0, CHD, chunk, 0)
    plsc.subcore_barrier()
    pltpu.sync_copy(accd.at[pl.ds(r0, RPT)], out_hbm.at[c, pl.ds(r0, RPT)])


@functools.partial(
    pl.kernel,
    out_type=jax.ShapeDtypeStruct((NSC, NPAD, DH), jnp.float32),
    mesh=_MESH,
    scratch_types=[
        pltpu.VMEM((CH, K), jnp.int32),
        pltpu.VMEM((CH, K), jnp.int32),
    ]
    + [pltpu.VMEM((K, DH), jnp.float32)] * NBUF
    + [pltpu.SemaphoreType.DMA] * (2 * NBUF)
    + [pltpu.VMEM_SHARED((NPAD, DH), jnp.float32)],
    compiler_params=pltpu.CompilerParams(use_tc_tiling_on_sc=False),
)
def _sc_hop(g2_hbm, src_hbm, dst_hbm, zero_hbm, out_hbm, src_v, dst_v, *ring):
    rows = ring[:NBUF]
    gs = ring[NBUF:2 * NBUF]
    ss = ring[2 * NBUF:3 * NBUF]
    acc = ring[3 * NBUF]
    c = lax.axis_index("c")
    s = lax.axis_index("s")
    w = c * NT + s
    r0 = s * RPT
    pltpu.sync_copy(zero_hbm.at[pl.ds(r0, RPT)], acc.at[pl.ds(r0, RPT)])
    pltpu.sync_copy(src_hbm.at[w], src_v)
    pltpu.sync_copy(dst_hbm.at[w], dst_v)
    plsc.subcore_barrier()

    for b in range(NBUF):
        pltpu.async_copy(g2_hbm.at[c].at[src_v.at[b]], rows[b], gs[b])

    def group(t, carry):
        base = t * NBUF
        for b in range(NBUF):
            i = base + b
            pltpu.make_async_copy(g2_hbm.at[c].at[src_v.at[i]], rows[b], gs[b]).wait()
            pltpu.async_copy(rows[b], acc.at[dst_v.at[i]], ss[b], add=True)
        for b in range(NBUF):
            i = base + b
            pltpu.make_async_copy(rows[b], acc.at[dst_v.at[i]], ss[b]).wait()
            pltpu.async_copy(g2_hbm.at[c].at[src_v.at[i + NBUF]], rows[b], gs[b])
        return carry

    lax.fori_loop(0, CH // NBUF - 1, group, 0)
    base = CH - NBUF
    for b in range(NBUF):
        i = base + b
        pltpu.make_async_copy(g2_hbm.at[c].at[src_v.at[i]], rows[b], gs[b]).wait()
        pltpu.async_copy(rows[b], acc.at[dst_v.at[i]], ss[b], add=True)
    for b in range(NBUF):
        i = base + b
        pltpu.make_async_copy(rows[b], acc.at[dst_v.at[i]], ss[b]).wait()

    plsc.subcore_barrier()
    pltpu.sync_copy(acc.at[pl.ds(r0, RPT)], out_hbm.at[c, pl.ds(r0, RPT)])


_R = 1000


def _tc_prep(degp, x):
    def body(degp_ref, x_ref, norm_ref, g2_ref):
        deg = degp_ref[0] + degp_ref[1]
        nb = jnp.where(deg[:, 0:1] > 0.0,
                       lax.rsqrt(jnp.maximum(deg[:, 0:1], 1.0)), 0.0)
        norm_ref[...] = nb
        xb = x_ref[...]
        g2_ref[0] = xb[:, 0:DH] * nb
        g2_ref[1] = xb[:, DH:D] * nb

    return pl.pallas_call(
        body,
        grid=(N // _R,),
        in_specs=[
            pl.BlockSpec((NSC, _R, WD), lambda r: (0, r, 0)),
            pl.BlockSpec((_R, D), lambda r: (r, 0)),
        ],
        out_specs=[
            pl.BlockSpec((_R, 1), lambda r: (r, 0)),
            pl.BlockSpec((NSC, _R, DH), lambda r: (0, r, 0)),
        ],
        out_shape=[
            jax.ShapeDtypeStruct((N, 1), jnp.float32),
            jax.ShapeDtypeStruct((NSC, N, DH), jnp.float32),
        ],
    )(degp, x)


def _tc_combine(p, norm_c):
    def body(p_ref, n_ref, c_ref, g2_ref):
        nb = n_ref[...]
        lo = p_ref[0] * nb
        hi = p_ref[1] * nb
        c_ref[...] = jnp.concatenate([lo, hi], axis=1)
        g2_ref[0] = lo * nb
        g2_ref[1] = hi * nb

    return pl.pallas_call(
        body,
        grid=(N // _R,),
        in_specs=[
            pl.BlockSpec((NSC, _R, DH), lambda r: (0, r, 0)),
            pl.BlockSpec((_R, 1), lambda r: (r, 0)),
        ],
        out_specs=[
            pl.BlockSpec((_R, D), lambda r: (r, 0)),
            pl.BlockSpec((NSC, _R, DH), lambda r: (0, r, 0)),
        ],
        out_shape=[
            jax.ShapeDtypeStruct((N, D), jnp.float32),
            jax.ShapeDtypeStruct((NSC, N, DH), jnp.float32),
        ],
    )(p, norm_c)


def _tc_layer1(x, c1, p, norm_c, W, b):
    def body(x_ref, c1_ref, p_ref, n_ref, w_ref, b_ref, h_ref, g2_ref):
        nb = n_ref[...]
        c2 = jnp.concatenate([p_ref[0], p_ref[1]], axis=1) * nb
        z = jnp.dot(x_ref[...], w_ref[0:D], preferred_element_type=jnp.float32)
        z += jnp.dot(c1_ref[...], w_ref[D:2 * D], preferred_element_type=jnp.float32)
        z += jnp.dot(c2, w_ref[2 * D:3 * D], preferred_element_type=jnp.float32)
        h = jnp.maximum(z + b_ref[...], 0.0)
        h_ref[...] = h
        hn = h * nb
        g2_ref[0] = hn[:, 0:DH]
        g2_ref[1] = hn[:, DH:D]

    return pl.pallas_call(
        body,
        grid=(N // _R,),
        in_specs=[
            pl.BlockSpec((_R, D), lambda r: (r, 0)),
            pl.BlockSpec((_R, D), lambda r: (r, 0)),
            pl.BlockSpec((NSC, _R, DH), lambda r: (0, r, 0)),
            pl.BlockSpec((_R, 1), lambda r: (r, 0)),
            pl.BlockSpec(((HOPS + 1) * D, HID), lambda r: (0, 0)),
            pl.BlockSpec((1, HID), lambda r: (0, 0)),
        ],
        out_specs=[
            pl.BlockSpec((_R, D), lambda r: (r, 0)),
            pl.BlockSpec((NSC, _R, DH), lambda r: (0, r, 0)),
        ],
        out_shape=[
            jax.ShapeDtypeStruct((N, HID), jnp.float32),
            jax.ShapeDtypeStruct((NSC, N, DH), jnp.float32),
        ],
    )(x, c1, p, norm_c, W, b)


def _tc_layer2(h1, c1, p, norm_c, W, b, Wc, bc):
    G = N // _R

    def body(h1_ref, c1_ref, p_ref, n_ref, w_ref, b_ref, wc_ref, bc_ref,
             out_ref, acc_ref):
        r = pl.program_id(0)
        nb = n_ref[...]
        c2 = jnp.concatenate([p_ref[0], p_ref[1]], axis=1) * nb
        z = jnp.dot(h1_ref[...], w_ref[0:D], preferred_element_type=jnp.float32)
        z += jnp.dot(c1_ref[...], w_ref[D:2 * D], preferred_element_type=jnp.float32)
        z += jnp.dot(c2, w_ref[2 * D:3 * D], preferred_element_type=jnp.float32)
        h = jnp.maximum(z + b_ref[...], 0.0)
        ssum = jnp.sum(h, axis=0, keepdims=True)

        @pl.when(r == 0)
        def _():
            acc_ref[...] = ssum

        @pl.when(r != 0)
        def _():
            acc_ref[...] = acc_ref[...] + ssum

        @pl.when(r == G - 1)
        def _():
            hg = acc_ref[...] * (1.0 / N)
            out_ref[...] = (
                jnp.dot(hg, wc_ref[...], preferred_element_type=jnp.float32)
                + bc_ref[...]
            )

    return pl.pallas_call(
        body,
        grid=(G,),
        in_specs=[
            pl.BlockSpec((_R, HID), lambda r: (r, 0)),
            pl.BlockSpec((_R, HID), lambda r: (r, 0)),
            pl.BlockSpec((NSC, _R, DH), lambda r: (0, r, 0)),
            pl.BlockSpec((_R, 1), lambda r: (r, 0)),
            pl.BlockSpec(((HOPS + 1) * HID, HID), lambda r: (0, 0)),
            pl.BlockSpec((1, HID), lambda r: (0, 0)),
            pl.BlockSpec((HID, NCLS), lambda r: (0, 0)),
            pl.BlockSpec((1, NCLS), lambda r: (0, 0)),
        ],
        out_specs=pl.BlockSpec((1, NCLS), lambda r: (0, 0)),
        out_shape=jax.ShapeDtypeStruct((1, NCLS), jnp.float32),
        scratch_shapes=[pltpu.VMEM((1, HID), jnp.float32)],
    )(h1, c1, p, norm_c, W, b, Wc, bc)


def kernel(x, edge_index, W1, b1, W2, b2, Wc, bc):
    src = edge_index[0].reshape(NW, CH, K)
    dst = edge_index[1].reshape(NW, CH, K)
    pad = NW * EPWP - E
    pad_dst = N + jnp.arange(pad, dtype=jnp.int32) % (NPAD - N)
    dstp = jnp.concatenate([edge_index[1], pad_dst]).reshape(NW, CHD, KD)
    zeros2d = jnp.zeros((NPAD, DH), jnp.float32)

    degp = _sc_deg(dstp, jnp.ones((KD, WD), jnp.float32),
                   jnp.zeros((NPAD, WD), jnp.float32))
    norm_c, g2 = _tc_prep(degp, x)

    pA = _sc_hop(g2, src, dst, zeros2d)
    c1, g2 = _tc_combine(pA, norm_c)
    pB = _sc_hop(g2, src, dst, zeros2d)
    h1, g2 = _tc_layer1(x, c1, pB, norm_c, W1, b1.reshape(1, HID))
    pC = _sc_hop(g2, src, dst, zeros2d)
    c1b, g2 = _tc_combine(pC, norm_c)
    pD = _sc_hop(g2, src, dst, zeros2d)
    return _tc_layer2(h1, c1b, pD, norm_c, W2, b2.reshape(1, HID),
                      Wc, bc.reshape(1, NCLS))
